# baseline (speedup 1.0000x reference)
import functools

import jax
import jax.numpy as jnp
from jax import lax
from jax.experimental import pallas as pl
from jax.experimental.pallas import tpu as pltpu
from jax.experimental.pallas import tpu_sc as plsc

N = 512
U = 128
D_IN = 128
IN_SZ = D_IN + U
NM = 3

LANES = 16
NW = 32
ROWS_PER_W = N // NW
TILE_WORDS = ROWS_PER_W * N


def _densify_body(nnz, nbuf, coo_hbm, out_hbm, coo_v, tile_v):
    wid = lax.axis_index("s") * 2 + lax.axis_index("c")
    lo = wid * ROWS_PER_W
    pltpu.sync_copy(coo_hbm, coo_v)

    zeros16 = jnp.zeros((LANES,), jnp.float32)

    def zero_body(i, _):
        tile_v[pl.ds(i * LANES, LANES)] = zeros16
        return 0

    lax.fori_loop(0, TILE_WORDS // LANES, zero_body, 0)

    def lower_bound(target):
        def cond(c):
            return c[0] < c[1]

        def body(c):
            lb, ub = c
            mid = (lb + ub) // 2
            v = coo_v[pl.ds(mid, LANES)][0]
            lt = v < target
            return jnp.where(lt, mid + 1, lb), jnp.where(lt, ub, mid)

        return lax.while_loop(cond, body, (jnp.int32(0), jnp.int32(nnz)))[0]

    e0 = lower_bound(lo)
    e1 = lower_bound(lo + ROWS_PER_W)
    start16 = (e0 // LANES) * LANES
    nvec = (e1 - start16 + LANES - 1) // LANES
    lane = lax.iota(jnp.int32, LANES)

    def edge_body(k, _):
        base = start16 + k * LANES
        r = coo_v[pl.ds(base, LANES)]
        c = coo_v[pl.ds(nbuf + base, LANES)]
        v = plsc.bitcast(coo_v[pl.ds(2 * nbuf + base, LANES)], jnp.float32)
        m = (r >= lo) & (r < lo + ROWS_PER_W) & (base + lane < nnz)
        lin = (r - lo) * N + c
        plsc.store_scatter(tile_v, [lin], v, mask=m)
        return 0

    lax.fori_loop(0, nvec, edge_body, 0)
    pltpu.sync_copy(tile_v, out_hbm.at[pl.ds(wid * TILE_WORDS, TILE_WORDS)])


def _densify(sup_rows, sup_cols, sup_vals):
    nnz = sup_rows.shape[0]
    nbuf = -(-nnz // 1024) * 1024
    pad = nbuf - nnz
    coo = jnp.concatenate(
        [
            sup_rows.astype(jnp.int32),
            jnp.full((pad,), 2 * N, jnp.int32),
            sup_cols.astype(jnp.int32),
            jnp.zeros((pad,), jnp.int32),
            jax.lax.bitcast_convert_type(sup_vals, jnp.int32),
            jnp.zeros((pad,), jnp.int32),
        ]
    )

    mesh = plsc.VectorSubcoreMesh(core_axis_name="c", subcore_axis_name="s")
    fn = functools.partial(
        pl.kernel,
        mesh=mesh,
        out_type=jax.ShapeDtypeStruct((N * N,), jnp.float32),
        scratch_types=[
            pltpu.VMEM((3 * nbuf,), jnp.int32),
            pltpu.VMEM((TILE_WORDS,), jnp.float32),
        ],
        compiler_params=pltpu.CompilerParams(needs_layout_passes=False),
    )(functools.partial(_densify_body, nnz, nbuf))
    return fn(coo).reshape(N, N)


def _dotf(a, b):
    return jnp.dot(a, b, preferred_element_type=jnp.float32)


def _dotb(a, b):
    return jnp.dot(a, b, preferred_element_type=jnp.float32).astype(jnp.bfloat16)


def _cell_kernel(s_ref, inp_ref, st_ref, wg_ref, bg_ref, wc_ref, bc_ref, out_ref, sbf_ref):
    @pl.when(pl.program_id(0) == 0)
    def _():
        sbf_ref[...] = s_ref[...].astype(jnp.bfloat16)

    s = sbf_ref[...]
    i0 = inp_ref[0].astype(jnp.bfloat16)
    i1 = inp_ref[1].astype(jnp.bfloat16)
    s0 = st_ref[0]
    s1 = st_ref[1]
    s0b = s0.astype(jnp.bfloat16)
    s1b = s1.astype(jnp.bfloat16)
    ist = jnp.concatenate([i0, i1, s0b, s1b], axis=1)
    d1 = _dotb(s, ist)
    d2 = _dotb(s, d1)
    g_in = jnp.concatenate(
        [
            jnp.concatenate([i0, s0b, d1[:, 0:128], d1[:, 256:384], d2[:, 0:128], d2[:, 256:384]], axis=1),
            jnp.concatenate([i1, s1b, d1[:, 128:256], d1[:, 384:512], d2[:, 128:256], d2[:, 384:512]], axis=1),
        ],
        axis=0,
    )
    g = jax.nn.sigmoid(_dotf(g_in, wg_ref[...]) + bg_ref[0])
    r0 = g[:N, :U]
    u0 = g[:N, U:]
    r1 = g[N:, :U]
    u1 = g[N:, U:]
    rs0 = (r0 * s0).astype(jnp.bfloat16)
    rs1 = (r1 * s1).astype(jnp.bfloat16)
    e1 = _dotb(s, jnp.concatenate([rs0, rs1], axis=1))
    e2 = _dotb(s, e1)
    c_in = jnp.concatenate(
        [
            jnp.concatenate([i0, rs0, d1[:, 0:128], e1[:, 0:128], d2[:, 0:128], e2[:, 0:128]], axis=1),
            jnp.concatenate([i1, rs1, d1[:, 128:256], e1[:, 128:256], d2[:, 128:256], e2[:, 128:256]], axis=1),
        ],
        axis=0,
    )
    c = jnp.tanh(_dotf(c_in, wc_ref[...]) + bc_ref[0])
    o0 = u0 * s0 + (1.0 - u0) * c[:N]
    o1 = u1 * s1 + (1.0 - u1) * c[N:]
    out_ref[0] = o0
    out_ref[1] = o1


def _prep_weights(w, out_sz):
    w3 = w.reshape(IN_SZ, NM, out_sz)
    return jnp.concatenate(
        [
            w3[:D_IN, 0] - w3[:D_IN, 2],
            w3[D_IN:, 0] - w3[D_IN:, 2],
            w3[:D_IN, 1],
            w3[D_IN:, 1],
            2.0 * w3[:D_IN, 2],
            2.0 * w3[D_IN:, 2],
        ],
        axis=0,
    )


def kernel(inputs, state, gate_weights, gate_biases, candidate_weights, candidate_biases, sup_rows, sup_cols, sup_vals):
    B = inputs.shape[0]
    BB = 2
    inp = inputs.reshape(B, N, D_IN)
    st = state.reshape(B, N, U)
    wg = _prep_weights(gate_weights, 2 * U)
    wc = _prep_weights(candidate_weights, U)
    bg = gate_biases.reshape(1, 2 * U)
    bc = candidate_biases.reshape(1, U)

    s_dense = _densify(sup_rows, sup_cols, sup_vals)

    out = pl.pallas_call(
        _cell_kernel,
        grid=(B // BB,),
        in_specs=[
            pl.BlockSpec((N, N), lambda b: (0, 0)),
            pl.BlockSpec((BB, N, D_IN), lambda b: (b, 0, 0)),
            pl.BlockSpec((BB, N, U), lambda b: (b, 0, 0)),
            pl.BlockSpec((NM * IN_SZ, 2 * U), lambda b: (0, 0)),
            pl.BlockSpec((1, 2 * U), lambda b: (0, 0)),
            pl.BlockSpec((NM * IN_SZ, U), lambda b: (0, 0)),
            pl.BlockSpec((1, U), lambda b: (0, 0)),
        ],
        out_specs=pl.BlockSpec((BB, N, U), lambda b: (b, 0, 0)),
        out_shape=jax.ShapeDtypeStruct((B, N, U), jnp.float32),
        scratch_shapes=[pltpu.VMEM((N, N), jnp.bfloat16)],
    )(s_dense, inp, st, wg.astype(jnp.bfloat16), bg, wc.astype(jnp.bfloat16), bc)

    y = out.reshape(B, N * U)
    return y, y


# speedup vs baseline: 1.0778x; 1.0778x over previous
import functools

import jax
import jax.numpy as jnp
from jax import lax
from jax.experimental import pallas as pl
from jax.experimental.pallas import tpu as pltpu
from jax.experimental.pallas import tpu_sc as plsc

N = 512
U = 128
D_IN = 128
IN_SZ = D_IN + U
NM = 3

LANES = 16
NW = 32
ROWS_PER_W = N // NW
TILE_WORDS = ROWS_PER_W * N


def _densify_body(nnz, nbuf, coo_hbm, out_hbm, coo_v, tile_v):
    wid = lax.axis_index("s") * 2 + lax.axis_index("c")
    lo = wid * ROWS_PER_W
    pltpu.sync_copy(coo_hbm, coo_v)

    zeros16 = jnp.zeros((LANES,), jnp.float32)

    def zero_body(i, _):
        tile_v[pl.ds(i * LANES, LANES)] = zeros16
        return 0

    lax.fori_loop(0, TILE_WORDS // LANES, zero_body, 0)

    def lower_bound(target):
        def cond(c):
            return c[0] < c[1]

        def body(c):
            lb, ub = c
            mid = (lb + ub) // 2
            v = coo_v[pl.ds(mid, LANES)][0]
            lt = v < target
            return jnp.where(lt, mid + 1, lb), jnp.where(lt, ub, mid)

        return lax.while_loop(cond, body, (jnp.int32(0), jnp.int32(nnz)))[0]

    e0 = lower_bound(lo)
    e1 = lower_bound(lo + ROWS_PER_W)
    start16 = (e0 // LANES) * LANES
    nvec = (e1 - start16 + LANES - 1) // LANES
    lane = lax.iota(jnp.int32, LANES)

    def edge_body(k, _):
        base = start16 + k * LANES
        r = coo_v[pl.ds(base, LANES)]
        c = coo_v[pl.ds(nbuf + base, LANES)]
        v = plsc.bitcast(coo_v[pl.ds(2 * nbuf + base, LANES)], jnp.float32)
        m = (r >= lo) & (r < lo + ROWS_PER_W) & (base + lane < nnz)
        lin = (r - lo) * N + c
        plsc.store_scatter(tile_v, [lin], v, mask=m)
        return 0

    lax.fori_loop(0, nvec, edge_body, 0)
    pltpu.sync_copy(tile_v, out_hbm.at[pl.ds(wid * TILE_WORDS, TILE_WORDS)])


def _densify(sup_rows, sup_cols, sup_vals):
    nnz = sup_rows.shape[0]
    nbuf = -(-nnz // 1024) * 1024
    pad = nbuf - nnz
    coo = jnp.concatenate(
        [
            sup_rows.astype(jnp.int32),
            jnp.full((pad,), 2 * N, jnp.int32),
            sup_cols.astype(jnp.int32),
            jnp.zeros((pad,), jnp.int32),
            jax.lax.bitcast_convert_type(sup_vals, jnp.int32),
            jnp.zeros((pad,), jnp.int32),
        ]
    )

    mesh = plsc.VectorSubcoreMesh(core_axis_name="c", subcore_axis_name="s")
    fn = functools.partial(
        pl.kernel,
        mesh=mesh,
        out_type=jax.ShapeDtypeStruct((N * N,), jnp.float32),
        scratch_types=[
            pltpu.VMEM((3 * nbuf,), jnp.int32),
            pltpu.VMEM((TILE_WORDS,), jnp.float32),
        ],
        compiler_params=pltpu.CompilerParams(needs_layout_passes=False),
    )(functools.partial(_densify_body, nnz, nbuf))
    return fn(coo).reshape(N, N)


def _dotf(a, b):
    return jnp.dot(a, b, preferred_element_type=jnp.float32)


def _dotb(a, b):
    return jnp.dot(a, b, preferred_element_type=jnp.float32).astype(jnp.bfloat16)


BB = 8


def _cell_kernel(s_ref, inp_ref, st_ref, wg_ref, bg_ref, wc_ref, bc_ref, out_ref, sbf_ref):
    @pl.when(pl.program_id(0) == 0)
    def _():
        sbf_ref[...] = s_ref[...].astype(jnp.bfloat16)

    s = sbf_ref[...]
    ii = [inp_ref[k].astype(jnp.bfloat16) for k in range(BB)]
    ss = [st_ref[k] for k in range(BB)]
    ssb = [x.astype(jnp.bfloat16) for x in ss]
    ist = jnp.concatenate(ii + ssb, axis=1)
    d1 = _dotb(s, ist)
    d2 = _dotb(s, d1)

    def sl(d, k):
        return d[:, k * 128 : (k + 1) * 128]

    g_in = jnp.concatenate(
        [
            jnp.concatenate([ii[k], ssb[k], sl(d1, k), sl(d1, BB + k), sl(d2, k), sl(d2, BB + k)], axis=1)
            for k in range(BB)
        ],
        axis=0,
    )
    g = jax.nn.sigmoid(_dotf(g_in, wg_ref[...]) + bg_ref[0])
    uu = [g[k * N : (k + 1) * N, U:] for k in range(BB)]
    rs = [(g[k * N : (k + 1) * N, :U] * ss[k]).astype(jnp.bfloat16) for k in range(BB)]
    e1 = _dotb(s, jnp.concatenate(rs, axis=1))
    e2 = _dotb(s, e1)
    c_in = jnp.concatenate(
        [
            jnp.concatenate([ii[k], rs[k], sl(d1, k), sl(e1, k), sl(d2, k), sl(e2, k)], axis=1)
            for k in range(BB)
        ],
        axis=0,
    )
    c = jnp.tanh(_dotf(c_in, wc_ref[...]) + bc_ref[0])
    for k in range(BB):
        out_ref[k] = uu[k] * ss[k] + (1.0 - uu[k]) * c[k * N : (k + 1) * N]


def _prep_weights(w, out_sz):
    w3 = w.reshape(IN_SZ, NM, out_sz)
    return jnp.concatenate(
        [
            w3[:D_IN, 0] - w3[:D_IN, 2],
            w3[D_IN:, 0] - w3[D_IN:, 2],
            w3[:D_IN, 1],
            w3[D_IN:, 1],
            2.0 * w3[:D_IN, 2],
            2.0 * w3[D_IN:, 2],
        ],
        axis=0,
    )


def kernel(inputs, state, gate_weights, gate_biases, candidate_weights, candidate_biases, sup_rows, sup_cols, sup_vals):
    B = inputs.shape[0]
    inp = inputs.reshape(B, N, D_IN)
    st = state.reshape(B, N, U)
    wg = _prep_weights(gate_weights, 2 * U)
    wc = _prep_weights(candidate_weights, U)
    bg = gate_biases.reshape(1, 2 * U)
    bc = candidate_biases.reshape(1, U)

    s_dense = _densify(sup_rows, sup_cols, sup_vals)

    out = pl.pallas_call(
        _cell_kernel,
        grid=(B // BB,),
        in_specs=[
            pl.BlockSpec((N, N), lambda b: (0, 0)),
            pl.BlockSpec((BB, N, D_IN), lambda b: (b, 0, 0)),
            pl.BlockSpec((BB, N, U), lambda b: (b, 0, 0)),
            pl.BlockSpec((NM * IN_SZ, 2 * U), lambda b: (0, 0)),
            pl.BlockSpec((1, 2 * U), lambda b: (0, 0)),
            pl.BlockSpec((NM * IN_SZ, U), lambda b: (0, 0)),
            pl.BlockSpec((1, U), lambda b: (0, 0)),
        ],
        out_specs=pl.BlockSpec((BB, N, U), lambda b: (b, 0, 0)),
        out_shape=jax.ShapeDtypeStruct((B, N, U), jnp.float32),
        scratch_shapes=[pltpu.VMEM((N, N), jnp.bfloat16)],
    )(s_dense, inp, st, wg.astype(jnp.bfloat16), bg, wc.astype(jnp.bfloat16), bc)

    y = out.reshape(B, N * U)
    return y, y


# speedup vs baseline: 1.0781x; 1.0003x over previous
import functools

import jax
import jax.numpy as jnp
from jax import lax
from jax.experimental import pallas as pl
from jax.experimental.pallas import tpu as pltpu
from jax.experimental.pallas import tpu_sc as plsc

N = 512
U = 128
D_IN = 128
IN_SZ = D_IN + U
NM = 3

LANES = 16
NW = 32
ROWS_PER_W = N // NW
TILE_WORDS = ROWS_PER_W * N


def _densify_body(nnz, nbuf, coo_hbm, out_hbm, coo_v, tile_v):
    wid = lax.axis_index("s") * 2 + lax.axis_index("c")
    lo = wid * ROWS_PER_W
    pltpu.sync_copy(coo_hbm, coo_v)

    zeros16 = jnp.zeros((LANES,), jnp.float32)

    def zero_body(i, _):
        tile_v[pl.ds(i * LANES, LANES)] = zeros16
        return 0

    lax.fori_loop(0, TILE_WORDS // LANES, zero_body, 0)

    def read16(pos):
        return coo_v[pos // 128, pl.ds((pos % 128) // LANES * LANES, LANES)]

    nv = -(-nnz // LANES)

    def first_ge(target):
        def cond(c):
            return c[0] < c[1]

        def body(c):
            lb, ub = c
            mid = (lb + ub) // 2
            v = read16(mid * LANES)[0]
            lt = v < target
            return jnp.where(lt, mid + 1, lb), jnp.where(lt, ub, mid)

        return lax.while_loop(cond, body, (jnp.int32(0), jnp.int32(nv)))[0]

    kstart = jnp.maximum(first_ge(lo) - 1, 0)
    kend = first_ge(lo + ROWS_PER_W)
    lane = lax.iota(jnp.int32, LANES)

    def edge_body(k, _):
        base = k * LANES
        r = read16(base)
        c = read16(nbuf + base)
        v = plsc.bitcast(read16(2 * nbuf + base), jnp.float32)
        m = (r >= lo) & (r < lo + ROWS_PER_W) & (base + lane < nnz)
        lin = (r - lo) * N + c
        plsc.store_scatter(tile_v, [lin], v, mask=m)
        return 0

    lax.fori_loop(kstart, kend, edge_body, 0)
    pltpu.sync_copy(tile_v, out_hbm.at[pl.ds(wid * TILE_WORDS, TILE_WORDS)])


def _densify(sup_rows, sup_cols, sup_vals):
    nnz = sup_rows.shape[0]
    nbuf = -(-nnz // 1024) * 1024
    pad = nbuf - nnz
    coo = jnp.concatenate(
        [
            sup_rows.astype(jnp.int32),
            jnp.full((pad,), 2 * N, jnp.int32),
            sup_cols.astype(jnp.int32),
            jnp.zeros((pad,), jnp.int32),
            jax.lax.bitcast_convert_type(sup_vals, jnp.int32),
            jnp.zeros((pad,), jnp.int32),
        ]
    ).reshape(3 * nbuf // 128, 128)

    mesh = plsc.VectorSubcoreMesh(core_axis_name="c", subcore_axis_name="s")
    fn = functools.partial(
        pl.kernel,
        mesh=mesh,
        out_type=jax.ShapeDtypeStruct((N * N,), jnp.float32),
        scratch_types=[
            pltpu.VMEM((3 * nbuf // 128, 128), jnp.int32),
            pltpu.VMEM((TILE_WORDS,), jnp.float32),
        ],
        compiler_params=pltpu.CompilerParams(needs_layout_passes=False),
    )(functools.partial(_densify_body, nnz, nbuf))
    return fn(coo).reshape(N, N)


def _dotf(a, b):
    return jnp.dot(a, b, preferred_element_type=jnp.float32)


def _dotb(a, b):
    return jnp.dot(a, b, preferred_element_type=jnp.float32).astype(jnp.bfloat16)


BB = 8


def _cell_kernel(s_ref, inp_ref, st_ref, wg_ref, bg_ref, wc_ref, bc_ref, out_ref, sbf_ref):
    @pl.when(pl.program_id(0) == 0)
    def _():
        sbf_ref[...] = s_ref[...].astype(jnp.bfloat16)

    s = sbf_ref[...]
    ii = [inp_ref[k].astype(jnp.bfloat16) for k in range(BB)]
    ss = [st_ref[k] for k in range(BB)]
    ssb = [x.astype(jnp.bfloat16) for x in ss]
    ist = jnp.concatenate(ii + ssb, axis=1)
    d1 = _dotb(s, ist)
    d2 = _dotb(s, d1)

    def sl(d, k):
        return d[:, k * 128 : (k + 1) * 128]

    g_in = jnp.concatenate(
        [
            jnp.concatenate([ii[k], ssb[k], sl(d1, k), sl(d1, BB + k), sl(d2, k), sl(d2, BB + k)], axis=1)
            for k in range(BB)
        ],
        axis=0,
    )
    g = jax.nn.sigmoid(_dotf(g_in, wg_ref[...]) + bg_ref[0])
    uu = [g[k * N : (k + 1) * N, U:] for k in range(BB)]
    rs = [(g[k * N : (k + 1) * N, :U] * ss[k]).astype(jnp.bfloat16) for k in range(BB)]
    e1 = _dotb(s, jnp.concatenate(rs, axis=1))
    e2 = _dotb(s, e1)
    c_in = jnp.concatenate(
        [
            jnp.concatenate([ii[k], rs[k], sl(d1, k), sl(e1, k), sl(d2, k), sl(e2, k)], axis=1)
            for k in range(BB)
        ],
        axis=0,
    )
    c = jnp.tanh(_dotf(c_in, wc_ref[...]) + bc_ref[0])
    for k in range(BB):
        out_ref[k] = uu[k] * ss[k] + (1.0 - uu[k]) * c[k * N : (k + 1) * N]


def _prep_weights(w, out_sz):
    w3 = w.reshape(IN_SZ, NM, out_sz)
    return jnp.concatenate(
        [
            w3[:D_IN, 0] - w3[:D_IN, 2],
            w3[D_IN:, 0] - w3[D_IN:, 2],
            w3[:D_IN, 1],
            w3[D_IN:, 1],
            2.0 * w3[:D_IN, 2],
            2.0 * w3[D_IN:, 2],
        ],
        axis=0,
    )


def kernel(inputs, state, gate_weights, gate_biases, candidate_weights, candidate_biases, sup_rows, sup_cols, sup_vals):
    B = inputs.shape[0]
    inp = inputs.reshape(B, N, D_IN)
    st = state.reshape(B, N, U)
    wg = _prep_weights(gate_weights, 2 * U)
    wc = _prep_weights(candidate_weights, U)
    bg = gate_biases.reshape(1, 2 * U)
    bc = candidate_biases.reshape(1, U)

    s_dense = _densify(sup_rows, sup_cols, sup_vals)

    out = pl.pallas_call(
        _cell_kernel,
        grid=(B // BB,),
        in_specs=[
            pl.BlockSpec((N, N), lambda b: (0, 0)),
            pl.BlockSpec((BB, N, D_IN), lambda b: (b, 0, 0)),
            pl.BlockSpec((BB, N, U), lambda b: (b, 0, 0)),
            pl.BlockSpec((NM * IN_SZ, 2 * U), lambda b: (0, 0)),
            pl.BlockSpec((1, 2 * U), lambda b: (0, 0)),
            pl.BlockSpec((NM * IN_SZ, U), lambda b: (0, 0)),
            pl.BlockSpec((1, U), lambda b: (0, 0)),
        ],
        out_specs=pl.BlockSpec((BB, N, U), lambda b: (b, 0, 0)),
        out_shape=jax.ShapeDtypeStruct((B, N, U), jnp.float32),
        scratch_shapes=[pltpu.VMEM((N, N), jnp.bfloat16)],
    )(s_dense, inp, st, wg.astype(jnp.bfloat16), bg, wc.astype(jnp.bfloat16), bc)

    y = out.reshape(B, N * U)
    return y, y


# speedup vs baseline: 1.5378x; 1.4264x over previous
import functools

import jax
import jax.numpy as jnp
from jax import lax
from jax.experimental import pallas as pl
from jax.experimental.pallas import tpu as pltpu
from jax.experimental.pallas import tpu_sc as plsc

N = 512
U = 128
D_IN = 128
IN_SZ = D_IN + U
NM = 3

LANES = 16
NW = 32
ROWS_PER_W = N // NW
TILE_WORDS = ROWS_PER_W * N


def _densify_body(nnz, nbuf, coo_hbm, out_hbm, coo_v, tile_v):
    wid = lax.axis_index("s") * 2 + lax.axis_index("c")
    lo = wid * ROWS_PER_W
    pltpu.sync_copy(coo_hbm, coo_v)

    zeros16 = jnp.zeros((LANES,), jnp.float32)

    def zero_body(i, _):
        tile_v[pl.ds(i * LANES, LANES)] = zeros16
        return 0

    lax.fori_loop(0, TILE_WORDS // LANES, zero_body, 0)

    def read16(pos):
        return coo_v[pos // 128, pl.ds((pos % 128) // LANES * LANES, LANES)]

    nv = -(-nnz // LANES)

    def first_ge(target):
        def cond(c):
            return c[0] < c[1]

        def body(c):
            lb, ub = c
            mid = (lb + ub) // 2
            v = read16(mid * LANES)[0]
            lt = v < target
            return jnp.where(lt, mid + 1, lb), jnp.where(lt, ub, mid)

        return lax.while_loop(cond, body, (jnp.int32(0), jnp.int32(nv)))[0]

    kstart = jnp.maximum(first_ge(lo) - 1, 0)
    kend = first_ge(lo + ROWS_PER_W)
    lane = lax.iota(jnp.int32, LANES)

    def edge_body(k, _):
        base = k * LANES
        r = read16(base)
        c = read16(nbuf + base)
        v = plsc.bitcast(read16(2 * nbuf + base), jnp.float32)
        m = (r >= lo) & (r < lo + ROWS_PER_W) & (base + lane < nnz)
        lin = (r - lo) * N + c
        plsc.store_scatter(tile_v, [lin], v, mask=m)
        return 0

    lax.fori_loop(kstart, kend, edge_body, 0)
    pltpu.sync_copy(tile_v, out_hbm.at[pl.ds(wid * TILE_WORDS, TILE_WORDS)])


def _densify(sup_rows, sup_cols, sup_vals):
    nnz = sup_rows.shape[0]
    nbuf = -(-nnz // 1024) * 1024
    pad = nbuf - nnz
    coo = jnp.concatenate(
        [
            sup_rows.astype(jnp.int32),
            jnp.full((pad,), 2 * N, jnp.int32),
            sup_cols.astype(jnp.int32),
            jnp.zeros((pad,), jnp.int32),
            jax.lax.bitcast_convert_type(sup_vals, jnp.int32),
            jnp.zeros((pad,), jnp.int32),
        ]
    ).reshape(3 * nbuf // 128, 128)

    mesh = plsc.VectorSubcoreMesh(core_axis_name="c", subcore_axis_name="s")
    fn = functools.partial(
        pl.kernel,
        mesh=mesh,
        out_type=jax.ShapeDtypeStruct((N * N,), jnp.float32),
        scratch_types=[
            pltpu.VMEM((3 * nbuf // 128, 128), jnp.int32),
            pltpu.VMEM((TILE_WORDS,), jnp.float32),
        ],
        compiler_params=pltpu.CompilerParams(needs_layout_passes=False),
    )(functools.partial(_densify_body, nnz, nbuf))
    return fn(coo).reshape(N, N)


def _dotf(a, b):
    return jnp.dot(a, b, preferred_element_type=jnp.float32)


def _dotb(a, b):
    return jnp.dot(a, b, preferred_element_type=jnp.float32).astype(jnp.bfloat16)


BB = 8


def _cell_kernel(s_ref, inp_ref, st_ref, wg_ref, bg_ref, wc_ref, bc_ref, out_ref, out2_ref, sbf_ref):
    @pl.when(pl.program_id(0) == 0)
    def _():
        sbf_ref[...] = s_ref[...].astype(jnp.bfloat16)

    s = sbf_ref[...]
    ii = [inp_ref[k].reshape(N, D_IN).astype(jnp.bfloat16) for k in range(BB)]
    ss = [st_ref[k].reshape(N, U) for k in range(BB)]
    ssb = [x.astype(jnp.bfloat16) for x in ss]
    ist = jnp.concatenate(ii + ssb, axis=1)
    d1 = _dotb(s, ist)
    d2 = _dotb(s, d1)

    def sl(d, k):
        return d[:, k * 128 : (k + 1) * 128]

    g_in = jnp.concatenate(
        [
            jnp.concatenate([ii[k], ssb[k], sl(d1, k), sl(d1, BB + k), sl(d2, k), sl(d2, BB + k)], axis=1)
            for k in range(BB)
        ],
        axis=0,
    )
    g = jax.nn.sigmoid(_dotf(g_in, wg_ref[...]) + bg_ref[0])
    uu = [g[k * N : (k + 1) * N, U:] for k in range(BB)]
    rs = [(g[k * N : (k + 1) * N, :U] * ss[k]).astype(jnp.bfloat16) for k in range(BB)]
    e1 = _dotb(s, jnp.concatenate(rs, axis=1))
    e2 = _dotb(s, e1)
    c_in = jnp.concatenate(
        [
            jnp.concatenate([ii[k], rs[k], sl(d1, k), sl(e1, k), sl(d2, k), sl(e2, k)], axis=1)
            for k in range(BB)
        ],
        axis=0,
    )
    c = jnp.tanh(_dotf(c_in, wc_ref[...]) + bc_ref[0])
    for k in range(BB):
        o = (uu[k] * ss[k] + (1.0 - uu[k]) * c[k * N : (k + 1) * N]).reshape(N * U)
        out_ref[k] = o
        out2_ref[k] = o


def _prep_weights(w, out_sz):
    w3 = w.reshape(IN_SZ, NM, out_sz)
    return jnp.concatenate(
        [
            w3[:D_IN, 0] - w3[:D_IN, 2],
            w3[D_IN:, 0] - w3[D_IN:, 2],
            w3[:D_IN, 1],
            w3[D_IN:, 1],
            2.0 * w3[:D_IN, 2],
            2.0 * w3[D_IN:, 2],
        ],
        axis=0,
    )


def kernel(inputs, state, gate_weights, gate_biases, candidate_weights, candidate_biases, sup_rows, sup_cols, sup_vals):
    B = inputs.shape[0]
    wg = _prep_weights(gate_weights, 2 * U)
    wc = _prep_weights(candidate_weights, U)
    bg = gate_biases.reshape(1, 2 * U)
    bc = candidate_biases.reshape(1, U)

    s_dense = _densify(sup_rows, sup_cols, sup_vals)

    out = pl.pallas_call(
        _cell_kernel,
        grid=(B // BB,),
        in_specs=[
            pl.BlockSpec((N, N), lambda b: (0, 0)),
            pl.BlockSpec((BB, N * D_IN), lambda b: (b, 0)),
            pl.BlockSpec((BB, N * U), lambda b: (b, 0)),
            pl.BlockSpec((NM * IN_SZ, 2 * U), lambda b: (0, 0)),
            pl.BlockSpec((1, 2 * U), lambda b: (0, 0)),
            pl.BlockSpec((NM * IN_SZ, U), lambda b: (0, 0)),
            pl.BlockSpec((1, U), lambda b: (0, 0)),
        ],
        out_specs=[
            pl.BlockSpec((BB, N * U), lambda b: (b, 0)),
            pl.BlockSpec((BB, N * U), lambda b: (b, 0)),
        ],
        out_shape=[
            jax.ShapeDtypeStruct((B, N * U), jnp.float32),
            jax.ShapeDtypeStruct((B, N * U), jnp.float32),
        ],
        scratch_shapes=[pltpu.VMEM((N, N), jnp.bfloat16)],
    )(s_dense, inputs, state, wg.astype(jnp.bfloat16), bg, wc.astype(jnp.bfloat16), bc)

    return out[0], out[1]


# speedup vs baseline: 1.6043x; 1.0433x over previous
import functools

import jax
import jax.numpy as jnp
from jax import lax
from jax.experimental import pallas as pl
from jax.experimental.pallas import tpu as pltpu
from jax.experimental.pallas import tpu_sc as plsc

N = 512
U = 128
D_IN = 128
IN_SZ = D_IN + U
NM = 3

LANES = 16
NW = 32
ROWS_PER_W = N // NW
TILE_WORDS = ROWS_PER_W * N


def _densify_body(nnz, sr_hbm, sc_hbm, sv_hbm, out_hbm, sr_v, sc_v, sv_v, tile_v):
    wid = lax.axis_index("s") * 2 + lax.axis_index("c")
    lo = wid * ROWS_PER_W
    pltpu.sync_copy(sr_hbm, sr_v.at[pl.ds(0, nnz)])
    pltpu.sync_copy(sc_hbm, sc_v.at[pl.ds(0, nnz)])
    pltpu.sync_copy(sv_hbm, sv_v.at[pl.ds(0, nnz)])

    zeros16 = jnp.zeros((LANES,), jnp.float32)

    def zero_body(i, _):
        tile_v[pl.ds(i * LANES, LANES)] = zeros16
        return 0

    lax.fori_loop(0, TILE_WORDS // LANES, zero_body, 0)

    def lower_bound(target):
        def cond(c):
            return c[0] < c[1]

        def body(c):
            lb, ub = c
            mid = (lb + ub) // 2
            v = sr_v[pl.ds(mid, LANES)][0]
            lt = v < target
            return jnp.where(lt, mid + 1, lb), jnp.where(lt, ub, mid)

        return lax.while_loop(cond, body, (jnp.int32(0), jnp.int32(nnz)))[0]

    e0 = lower_bound(lo)
    e1 = lower_bound(lo + ROWS_PER_W)
    start16 = (e0 // LANES) * LANES
    nvec = (e1 - start16 + LANES - 1) // LANES
    lane = lax.iota(jnp.int32, LANES)

    def edge_body(k, _):
        base = start16 + k * LANES
        r = sr_v[pl.ds(base, LANES)]
        c = sc_v[pl.ds(base, LANES)]
        v = sv_v[pl.ds(base, LANES)]
        m = (r >= lo) & (r < lo + ROWS_PER_W) & (base + lane < nnz)
        lin = (r - lo) * N + c
        plsc.store_scatter(tile_v, [lin], v, mask=m)
        return 0

    lax.fori_loop(0, nvec, edge_body, 0)
    pltpu.sync_copy(tile_v, out_hbm.at[pl.ds(wid * TILE_WORDS, TILE_WORDS)])


def _densify(sup_rows, sup_cols, sup_vals):
    nnz = sup_rows.shape[0]
    nbuf = (nnz // LANES + 2) * LANES

    mesh = plsc.VectorSubcoreMesh(core_axis_name="c", subcore_axis_name="s")
    fn = functools.partial(
        pl.kernel,
        mesh=mesh,
        out_type=jax.ShapeDtypeStruct((N * N,), jnp.float32),
        scratch_types=[
            pltpu.VMEM((nbuf,), jnp.int32),
            pltpu.VMEM((nbuf,), jnp.int32),
            pltpu.VMEM((nbuf,), jnp.float32),
            pltpu.VMEM((TILE_WORDS,), jnp.float32),
        ],
        compiler_params=pltpu.CompilerParams(needs_layout_passes=False),
    )(functools.partial(_densify_body, nnz))
    return fn(sup_rows.astype(jnp.int32), sup_cols.astype(jnp.int32), sup_vals).reshape(N, N)


def _dotf(a, b):
    return jnp.dot(a, b, preferred_element_type=jnp.float32)


def _dotb(a, b):
    return jnp.dot(a, b, preferred_element_type=jnp.float32).astype(jnp.bfloat16)


BB = 8


def _cell_kernel(s_ref, inp_ref, st_ref, wg_ref, bg_ref, wc_ref, bc_ref, out_ref, out2_ref, sbf_ref):
    @pl.when(pl.program_id(0) == 0)
    def _():
        sbf_ref[...] = s_ref[...].astype(jnp.bfloat16)

    s = sbf_ref[...]
    ii = [inp_ref[k].reshape(N, D_IN).astype(jnp.bfloat16) for k in range(BB)]
    ss = [st_ref[k].reshape(N, U) for k in range(BB)]
    ssb = [x.astype(jnp.bfloat16) for x in ss]
    ist = jnp.concatenate(ii + ssb, axis=1)
    d1 = _dotb(s, ist)
    d2 = _dotb(s, d1)

    def sl(d, k):
        return d[:, k * 128 : (k + 1) * 128]

    g_in = jnp.concatenate(
        [
            jnp.concatenate([ii[k], ssb[k], sl(d1, k), sl(d1, BB + k), sl(d2, k), sl(d2, BB + k)], axis=1)
            for k in range(BB)
        ],
        axis=0,
    )
    g = jax.nn.sigmoid(_dotf(g_in, wg_ref[...]) + bg_ref[0])
    uu = [g[k * N : (k + 1) * N, U:] for k in range(BB)]
    rs = [(g[k * N : (k + 1) * N, :U] * ss[k]).astype(jnp.bfloat16) for k in range(BB)]
    e1 = _dotb(s, jnp.concatenate(rs, axis=1))
    e2 = _dotb(s, e1)
    c_in = jnp.concatenate(
        [
            jnp.concatenate([ii[k], rs[k], sl(d1, k), sl(e1, k), sl(d2, k), sl(e2, k)], axis=1)
            for k in range(BB)
        ],
        axis=0,
    )
    c = jnp.tanh(_dotf(c_in, wc_ref[...]) + bc_ref[0])
    for k in range(BB):
        o = (uu[k] * ss[k] + (1.0 - uu[k]) * c[k * N : (k + 1) * N]).reshape(N * U)
        out_ref[k] = o
        out2_ref[k] = o


def _prep_weights(w, out_sz):
    w3 = w.reshape(IN_SZ, NM, out_sz)
    return jnp.concatenate(
        [
            w3[:D_IN, 0] - w3[:D_IN, 2],
            w3[D_IN:, 0] - w3[D_IN:, 2],
            w3[:D_IN, 1],
            w3[D_IN:, 1],
            2.0 * w3[:D_IN, 2],
            2.0 * w3[D_IN:, 2],
        ],
        axis=0,
    )


def kernel(inputs, state, gate_weights, gate_biases, candidate_weights, candidate_biases, sup_rows, sup_cols, sup_vals):
    B = inputs.shape[0]
    wg = _prep_weights(gate_weights, 2 * U)
    wc = _prep_weights(candidate_weights, U)
    bg = gate_biases.reshape(1, 2 * U)
    bc = candidate_biases.reshape(1, U)

    s_dense = _densify(sup_rows, sup_cols, sup_vals)

    out = pl.pallas_call(
        _cell_kernel,
        grid=(B // BB,),
        in_specs=[
            pl.BlockSpec((N, N), lambda b: (0, 0)),
            pl.BlockSpec((BB, N * D_IN), lambda b: (b, 0)),
            pl.BlockSpec((BB, N * U), lambda b: (b, 0)),
            pl.BlockSpec((NM * IN_SZ, 2 * U), lambda b: (0, 0)),
            pl.BlockSpec((1, 2 * U), lambda b: (0, 0)),
            pl.BlockSpec((NM * IN_SZ, U), lambda b: (0, 0)),
            pl.BlockSpec((1, U), lambda b: (0, 0)),
        ],
        out_specs=[
            pl.BlockSpec((BB, N * U), lambda b: (b, 0)),
            pl.BlockSpec((BB, N * U), lambda b: (b, 0)),
        ],
        out_shape=[
            jax.ShapeDtypeStruct((B, N * U), jnp.float32),
            jax.ShapeDtypeStruct((B, N * U), jnp.float32),
        ],
        scratch_shapes=[pltpu.VMEM((N, N), jnp.bfloat16)],
    )(s_dense, inputs, state, wg.astype(jnp.bfloat16), bg, wc.astype(jnp.bfloat16), bc)

    return out[0], out[1]
